# initial kernel scaffold (unmeasured)
import jax
import jax.numpy as jnp
from jax import lax
from jax.experimental import pallas as pl
from jax.experimental.pallas import tpu as pltpu

NDEV = 4
KBLK = 1024

_CompilerParams = getattr(pltpu, "CompilerParams", None) or getattr(
    pltpu, "TPUCompilerParams"
)


def _gelu(y):
    c = 0.7978845608028654
    return 0.5 * y * (1.0 + jnp.tanh(c * (y + 0.044715 * y * y * y)))


def kernel(x, w_mat):
    m_per, k_tot = x.shape
    n_tot = w_mat.shape[1]
    n_per = n_tot // NDEV
    m_tot = m_per * NDEV
    nk = k_tot // KBLK

    def body(x_ref, w_ref, out_ref, acc_ref, send_sems, recv_sems):
        p = pl.program_id(0)
        k = pl.program_id(1)
        me = lax.axis_index("i")
        j = (me + p + 1) % NDEV
        slot = p % 2

        @pl.when((p == 0) & (k == 0))
        def _entry_barrier():
            bsem = pltpu.get_barrier_semaphore()
            for off in range(1, NDEV):
                pl.semaphore_signal(
                    bsem,
                    inc=1,
                    device_id=((me + off) % NDEV,),
                    device_id_type=pl.DeviceIdType.MESH,
                )
            pl.semaphore_wait(bsem, NDEV - 1)

        @pl.when((k == 0) & (p >= 2))
        def _drain_prev_send():
            pltpu.make_async_remote_copy(
                src_ref=acc_ref.at[slot],
                dst_ref=out_ref.at[pl.ds(0, m_per), :],
                send_sem=send_sems.at[p - 2],
                recv_sem=recv_sems.at[me],
                device_id=((me + 1) % NDEV,),
                device_id_type=pl.DeviceIdType.MESH,
            ).wait_send()

        part = jnp.dot(
            x_ref[...], w_ref[...], preferred_element_type=jnp.float32
        )

        @pl.when(k == 0)
        def _init():
            acc_ref[slot] = part

        @pl.when(k > 0)
        def _accum():
            acc_ref[slot] = acc_ref[slot] + part

        @pl.when(k == nk - 1)
        def _finish_phase():
            yg = _gelu(acc_ref[slot])

            @pl.when(j == me)
            def _local():
                out_ref[pl.ds(me * m_per, m_per), :] = yg

            @pl.when(j != me)
            def _remote():
                acc_ref[slot] = yg
                pltpu.make_async_remote_copy(
                    src_ref=acc_ref.at[slot],
                    dst_ref=out_ref.at[pl.ds(me * m_per, m_per), :],
                    send_sem=send_sems.at[p],
                    recv_sem=recv_sems.at[me],
                    device_id=(j,),
                    device_id_type=pl.DeviceIdType.MESH,
                ).start()

        @pl.when((p == NDEV - 1) & (k == nk - 1))
        def _finish_kernel():
            pltpu.make_async_remote_copy(
                src_ref=acc_ref.at[0],
                dst_ref=out_ref.at[pl.ds(0, m_per), :],
                send_sem=send_sems.at[2],
                recv_sem=recv_sems.at[me],
                device_id=((me + 1) % NDEV,),
                device_id_type=pl.DeviceIdType.MESH,
            ).wait_send()
            for off in range(1, NDEV):
                s = (me + off) % NDEV
                pltpu.make_async_remote_copy(
                    src_ref=acc_ref.at[0],
                    dst_ref=out_ref.at[pl.ds(s * m_per, m_per), :],
                    send_sem=send_sems.at[NDEV - 1],
                    recv_sem=recv_sems.at[s],
                    device_id=(s,),
                    device_id_type=pl.DeviceIdType.MESH,
                ).wait_recv()

    return pl.pallas_call(
        body,
        grid=(NDEV, nk),
        in_specs=[
            pl.BlockSpec((m_per, KBLK), lambda p, k: (0, k)),
            pl.BlockSpec(
                (KBLK, n_per),
                lambda p, k: (k, (lax.axis_index("i") + p + 1) % NDEV),
            ),
        ],
        out_specs=pl.BlockSpec((m_tot, n_per), lambda p, k: (0, 0)),
        out_shape=jax.ShapeDtypeStruct((m_tot, n_per), jnp.float32),
        scratch_shapes=[
            pltpu.VMEM((2, m_per, n_per), jnp.float32),
            pltpu.SemaphoreType.DMA((NDEV,)),
            pltpu.SemaphoreType.DMA((NDEV,)),
        ],
        compiler_params=_CompilerParams(
            collective_id=0,
            dimension_semantics=("arbitrary", "arbitrary"),
            vmem_limit_bytes=128 * 1024 * 1024,
        ),
    )(x, w_mat)


# baseline (device time: 325232 ns/iter reference)
import jax
import jax.numpy as jnp
from jax import lax
from jax.experimental import pallas as pl
from jax.experimental.pallas import tpu as pltpu

NDEV = 4
KBLK = 512

_CompilerParams = getattr(pltpu, "CompilerParams", None) or getattr(
    pltpu, "TPUCompilerParams"
)


def _gelu(y):
    c = 0.7978845608028654
    return 0.5 * y * (1.0 + jnp.tanh(c * (y + 0.044715 * y * y * y)))


def kernel(x, w_mat):
    m_per, k_tot = x.shape
    n_tot = w_mat.shape[1]
    n_per = n_tot // NDEV
    m_tot = m_per * NDEV
    nk = k_tot // KBLK

    def body(x_ref, w_ref, out_ref, acc_ref, send_sems, recv_sems, local_sem):
        p = pl.program_id(0)
        k = pl.program_id(1)
        me = lax.axis_index("i")
        j = (me + p + 1) % NDEV
        slot = p % 2

        @pl.when((p == 0) & (k == 0))
        def _entry_barrier():
            bsem = pltpu.get_barrier_semaphore()
            for off in range(1, NDEV):
                pl.semaphore_signal(
                    bsem,
                    inc=1,
                    device_id=((me + off) % NDEV,),
                    device_id_type=pl.DeviceIdType.MESH,
                )
            pl.semaphore_wait(bsem, NDEV - 1)

        @pl.when((k == 0) & (p >= 2))
        def _drain_prev_send():
            pltpu.make_async_remote_copy(
                src_ref=acc_ref.at[slot],
                dst_ref=out_ref.at[pl.ds(0, m_per), :],
                send_sem=send_sems.at[p - 2],
                recv_sem=recv_sems.at[me],
                device_id=((me + 1) % NDEV,),
                device_id_type=pl.DeviceIdType.MESH,
            ).wait_send()

        part = jnp.dot(
            x_ref[...], w_ref[...], preferred_element_type=jnp.float32
        )

        @pl.when(k == 0)
        def _init():
            acc_ref[slot] = part

        @pl.when(k > 0)
        def _accum():
            acc_ref[slot] = acc_ref[slot] + part

        @pl.when(k == nk - 1)
        def _finish_phase():
            yg = _gelu(acc_ref[slot])
            acc_ref[slot] = yg

            @pl.when(j == me)
            def _local():
                pltpu.make_async_copy(
                    acc_ref.at[slot],
                    out_ref.at[pl.ds(me * m_per, m_per), :],
                    local_sem,
                ).start()

            @pl.when(j != me)
            def _remote():
                pltpu.make_async_remote_copy(
                    src_ref=acc_ref.at[slot],
                    dst_ref=out_ref.at[pl.ds(me * m_per, m_per), :],
                    send_sem=send_sems.at[p],
                    recv_sem=recv_sems.at[me],
                    device_id=(j,),
                    device_id_type=pl.DeviceIdType.MESH,
                ).start()

        @pl.when((p == NDEV - 1) & (k == nk - 1))
        def _finish_kernel():
            pltpu.make_async_copy(
                acc_ref.at[slot],
                out_ref.at[pl.ds(me * m_per, m_per), :],
                local_sem,
            ).wait()
            pltpu.make_async_remote_copy(
                src_ref=acc_ref.at[0],
                dst_ref=out_ref.at[pl.ds(0, m_per), :],
                send_sem=send_sems.at[2],
                recv_sem=recv_sems.at[me],
                device_id=((me + 1) % NDEV,),
                device_id_type=pl.DeviceIdType.MESH,
            ).wait_send()
            for off in range(1, NDEV):
                s = (me + off) % NDEV
                pltpu.make_async_remote_copy(
                    src_ref=acc_ref.at[0],
                    dst_ref=out_ref.at[pl.ds(s * m_per, m_per), :],
                    send_sem=send_sems.at[NDEV - 1],
                    recv_sem=recv_sems.at[s],
                    device_id=(s,),
                    device_id_type=pl.DeviceIdType.MESH,
                ).wait_recv()

    return pl.pallas_call(
        body,
        grid=(NDEV, nk),
        in_specs=[
            pl.BlockSpec((m_per, KBLK), lambda p, k: (0, k)),
            pl.BlockSpec(
                (KBLK, n_per),
                lambda p, k: (k, (lax.axis_index("i") + p + 1) % NDEV),
            ),
        ],
        out_specs=pl.BlockSpec(memory_space=pl.ANY),
        out_shape=jax.ShapeDtypeStruct((m_tot, n_per), jnp.float32),
        scratch_shapes=[
            pltpu.VMEM((2, m_per, n_per), jnp.float32),
            pltpu.SemaphoreType.DMA((NDEV,)),
            pltpu.SemaphoreType.DMA((NDEV,)),
            pltpu.SemaphoreType.DMA,
        ],
        compiler_params=_CompilerParams(
            collective_id=0,
            dimension_semantics=("arbitrary", "arbitrary"),
            vmem_limit_bytes=60 * 1024 * 1024,
        ),
    )(x, w_mat)


# device time: 323878 ns/iter; 1.0042x vs baseline; 1.0042x over previous
import jax
import jax.numpy as jnp
from jax import lax
from jax.experimental import pallas as pl
from jax.experimental.pallas import tpu as pltpu

NDEV = 4
KBLK = 512

_CompilerParams = getattr(pltpu, "CompilerParams", None) or getattr(
    pltpu, "TPUCompilerParams"
)


def _gelu(y):
    c = 0.7978845608028654
    return 0.5 * y * (1.0 + jnp.tanh(c * (y + 0.044715 * y * y * y)))


def kernel(x, w_mat):
    m_per, k_tot = x.shape
    n_tot = w_mat.shape[1]
    n_per = n_tot // NDEV
    m_tot = m_per * NDEV
    nk = k_tot // KBLK

    def body(x_ref, w_ref, out_ref, acc_ref, send_sems, recv_sems, local_sem):
        p = pl.program_id(0)
        k = pl.program_id(1)
        me = lax.axis_index("i")
        j = (me + p + 1) % NDEV
        slot = p

        @pl.when((p == 0) & (k == 0))
        def _entry_barrier():
            bsem = pltpu.get_barrier_semaphore()
            for off in range(1, NDEV):
                pl.semaphore_signal(
                    bsem,
                    inc=1,
                    device_id=((me + off) % NDEV,),
                    device_id_type=pl.DeviceIdType.MESH,
                )
            pl.semaphore_wait(bsem, NDEV - 1)

        part = jnp.dot(
            x_ref[...], w_ref[...], preferred_element_type=jnp.float32
        )

        @pl.when(k == 0)
        def _init():
            acc_ref[slot] = part

        @pl.when(k > 0)
        def _accum():
            acc_ref[slot] = acc_ref[slot] + part

        @pl.when(k == nk - 1)
        def _finish_phase():
            yg = _gelu(acc_ref[slot])
            acc_ref[slot] = yg

            @pl.when(j == me)
            def _local():
                pltpu.make_async_copy(
                    acc_ref.at[slot],
                    out_ref.at[pl.ds(me * m_per, m_per), :],
                    local_sem,
                ).start()

            @pl.when(j != me)
            def _remote():
                pltpu.make_async_remote_copy(
                    src_ref=acc_ref.at[slot],
                    dst_ref=out_ref.at[pl.ds(me * m_per, m_per), :],
                    send_sem=send_sems.at[p],
                    recv_sem=recv_sems.at[me],
                    device_id=(j,),
                    device_id_type=pl.DeviceIdType.MESH,
                ).start()

        @pl.when((p == NDEV - 1) & (k == nk - 1))
        def _finish_kernel():
            pltpu.make_async_copy(
                acc_ref.at[slot],
                out_ref.at[pl.ds(me * m_per, m_per), :],
                local_sem,
            ).wait()
            for q in range(NDEV - 1):
                pltpu.make_async_remote_copy(
                    src_ref=acc_ref.at[q],
                    dst_ref=out_ref.at[pl.ds(0, m_per), :],
                    send_sem=send_sems.at[q],
                    recv_sem=recv_sems.at[me],
                    device_id=((me + 1) % NDEV,),
                    device_id_type=pl.DeviceIdType.MESH,
                ).wait_send()
            for off in range(1, NDEV):
                s = (me + off) % NDEV
                pltpu.make_async_remote_copy(
                    src_ref=acc_ref.at[0],
                    dst_ref=out_ref.at[pl.ds(s * m_per, m_per), :],
                    send_sem=send_sems.at[NDEV - 1],
                    recv_sem=recv_sems.at[s],
                    device_id=(s,),
                    device_id_type=pl.DeviceIdType.MESH,
                ).wait_recv()

    return pl.pallas_call(
        body,
        grid=(NDEV, nk),
        in_specs=[
            pl.BlockSpec((m_per, KBLK), lambda p, k: (0, k)),
            pl.BlockSpec(
                (KBLK, n_per),
                lambda p, k: (k, (lax.axis_index("i") + p + 1) % NDEV),
            ),
        ],
        out_specs=pl.BlockSpec(memory_space=pl.ANY),
        out_shape=jax.ShapeDtypeStruct((m_tot, n_per), jnp.float32),
        scratch_shapes=[
            pltpu.VMEM((NDEV, m_per, n_per), jnp.float32),
            pltpu.SemaphoreType.DMA((NDEV,)),
            pltpu.SemaphoreType.DMA((NDEV,)),
            pltpu.SemaphoreType.DMA,
        ],
        compiler_params=_CompilerParams(
            collective_id=0,
            dimension_semantics=("arbitrary", "arbitrary"),
            vmem_limit_bytes=60 * 1024 * 1024,
        ),
    )(x, w_mat)


# device time: 254638 ns/iter; 1.2772x vs baseline; 1.2719x over previous
import jax
import jax.numpy as jnp
from jax import lax
from jax.experimental import pallas as pl
from jax.experimental.pallas import tpu as pltpu

NDEV = 4
KBLK = 1024

_CompilerParams = getattr(pltpu, "CompilerParams", None) or getattr(
    pltpu, "TPUCompilerParams"
)


def _t_of_p(p):
    return jnp.where(p == 0, 2, jnp.where(p == 1, 1, jnp.where(p == 2, 3, 0)))


def _gelu(y):
    c = 0.7978845608028654
    return 0.5 * y * (1.0 + jnp.tanh(c * (y + 0.044715 * y * y * y)))


def kernel(x, w_mat):
    m_per, k_tot = x.shape
    n_tot = w_mat.shape[1]
    n_per = n_tot // NDEV
    m_tot = m_per * NDEV
    nk = k_tot // KBLK

    def body(x_ref, w_ref, out_ref, rstage, acc_ref, sstage,
             send_sems, recv_sems, local_sem):
        p = pl.program_id(0)
        k = pl.program_id(1)
        me = lax.axis_index("i")
        t = _t_of_p(p)
        j = (me + t) % NDEV

        @pl.when((p == 0) & (k == 0))
        def _entry_barrier():
            bsem = pltpu.get_barrier_semaphore()
            for off in range(1, NDEV):
                pl.semaphore_signal(
                    bsem,
                    inc=1,
                    device_id=((me + off) % NDEV,),
                    device_id_type=pl.DeviceIdType.MESH,
                )
            pl.semaphore_wait(bsem, NDEV - 1)

        part = jnp.dot(
            x_ref[...], w_ref[...], preferred_element_type=jnp.float32
        )

        @pl.when(k == 0)
        def _init():
            acc_ref[...] = jnp.zeros_like(part)

        acc_ref[...] += part

        @pl.when(k == nk - 1)
        def _finish_phase():
            y = _gelu(acc_ref[...])

            @pl.when(j != me)
            def _remote():
                @pl.when(p >= 1)
                def _drain_prev():
                    pltpu.make_async_remote_copy(
                        src_ref=sstage,
                        dst_ref=rstage.at[0],
                        send_sem=send_sems.at[p - 1],
                        recv_sem=recv_sems.at[0],
                        device_id=((me + 1) % NDEV,),
                        device_id_type=pl.DeviceIdType.MESH,
                    ).wait_send()

                sstage[...] = y.astype(jnp.bfloat16)
                pltpu.make_async_remote_copy(
                    src_ref=sstage,
                    dst_ref=rstage.at[p],
                    send_sem=send_sems.at[p],
                    recv_sem=recv_sems.at[p],
                    device_id=(j,),
                    device_id_type=pl.DeviceIdType.MESH,
                ).start()

            @pl.when(j == me)
            def _local():
                acc_ref[...] = y
                pltpu.make_async_copy(
                    acc_ref,
                    out_ref.at[pl.ds(me * m_per, m_per), :],
                    local_sem,
                ).start()

        @pl.when((p == NDEV - 1) & (k == nk - 1))
        def _finish_kernel():
            pltpu.make_async_copy(
                acc_ref,
                out_ref.at[pl.ds(me * m_per, m_per), :],
                local_sem,
            ).wait()
            pltpu.make_async_remote_copy(
                src_ref=sstage,
                dst_ref=rstage.at[0],
                send_sem=send_sems.at[NDEV - 2],
                recv_sem=recv_sems.at[0],
                device_id=((me + 1) % NDEV,),
                device_id_type=pl.DeviceIdType.MESH,
            ).wait_send()
            for slot, t_s in enumerate((2, 1, 3)):
                s = (me - t_s) % NDEV
                pltpu.make_async_remote_copy(
                    src_ref=sstage,
                    dst_ref=rstage.at[slot],
                    send_sem=send_sems.at[NDEV - 1],
                    recv_sem=recv_sems.at[slot],
                    device_id=(s,),
                    device_id_type=pl.DeviceIdType.MESH,
                ).wait_recv()
                bounce = pltpu.make_async_copy(
                    rstage.at[slot], sstage, local_sem
                )
                bounce.start()
                bounce.wait()
                acc_ref[...] = sstage[...].astype(jnp.float32)
                cp = pltpu.make_async_copy(
                    acc_ref,
                    out_ref.at[pl.ds(s * m_per, m_per), :],
                    local_sem,
                )
                cp.start()
                cp.wait()

    out, _ = pl.pallas_call(
        body,
        grid=(NDEV, nk),
        in_specs=[
            pl.BlockSpec((m_per, KBLK), lambda p, k: (0, k)),
            pl.BlockSpec(
                (KBLK, n_per),
                lambda p, k: (k, (lax.axis_index("i") + _t_of_p(p)) % NDEV),
            ),
        ],
        out_specs=(
            pl.BlockSpec(memory_space=pl.ANY),
            pl.BlockSpec(memory_space=pl.ANY),
        ),
        out_shape=(
            jax.ShapeDtypeStruct((m_tot, n_per), jnp.float32),
            jax.ShapeDtypeStruct((NDEV - 1, m_per, n_per), jnp.bfloat16),
        ),
        scratch_shapes=[
            pltpu.VMEM((m_per, n_per), jnp.float32),
            pltpu.VMEM((m_per, n_per), jnp.bfloat16),
            pltpu.SemaphoreType.DMA((NDEV,)),
            pltpu.SemaphoreType.DMA((NDEV,)),
            pltpu.SemaphoreType.DMA,
        ],
        compiler_params=_CompilerParams(
            collective_id=0,
            dimension_semantics=("arbitrary", "arbitrary"),
            vmem_limit_bytes=60 * 1024 * 1024,
        ),
    )(x, w_mat)
    return out
